# Initial kernel scaffold; baseline (speedup 1.0000x reference)
import functools

import jax
import jax.numpy as jnp
import numpy as np
from jax import lax
from jax.experimental import pallas as pl
from jax.experimental.pallas import tpu as pltpu
from jax.experimental.pallas import tpu_sc as plsc

_N = 10000
_E = 320000
_F = 128
_H = 8
_D = 16

_NC = 2
_NS = 16
_C = 80

_BN = 400
_BE = 512


def _dot(a, b):
    return jnp.dot(a, b, preferred_element_type=jnp.float32)


def _node_body(x_ref, wq_ref, wk_ref, wv_ref, wrkb_ref, wrvb_ref, m_ref,
               vecs_ref, q4_ref, bk_ref, bv_ref):
    x = x_ref[...]
    M = m_ref[...]
    V = vecs_ref

    def ln(y, gi, bi):
        mu = _dot(y, M)
        ex2 = _dot(y * y, M)
        r = lax.rsqrt(ex2 - mu * mu + 1e-5)
        return (y - mu) * r * V[gi:gi + 1, :] + V[bi:bi + 1, :]

    q = ln(_dot(x, wq_ref[...]), 0, 1)
    q4_ref[...] = q * 0.25
    k = ln(_dot(x, wk_ref[...]), 2, 3)
    bk_ref[...] = _dot(k, wrkb_ref[...])
    v = ln(_dot(x, wv_ref[...]), 4, 5)
    bv_ref[...] = _dot(v, wrvb_ref[...])


def _edge_body(xe_ref, gq_ref, gbk_ref, gbv_ref, wke_ref, wve_ref, wrkt_ref,
               wrvt_ref, m_ref, ssum_ref, sp8_ref, vecs_ref,
               p0_ref, p1_ref, eq_ref):
    x = xe_ref[...]
    M = m_ref[...]
    V = vecs_ref

    def ln(y, gi, bi):
        mu = _dot(y, M)
        ex2 = _dot(y * y, M)
        r = lax.rsqrt(ex2 - mu * mu + 1e-5)
        return (y - mu) * r * V[gi:gi + 1, :] + V[bi:bi + 1, :]

    ke = ln(_dot(x, wke_ref[...]), 0, 1)
    ak = _dot(ke, wrkt_ref[...]) + V[8:9, :]
    relk = ln(ak + gbk_ref[...], 4, 5)
    prod = gq_ref[...] * relk
    qkb = _dot(prod, ssum_ref[...])
    eq_ref[...] = jnp.exp(_dot(prod, sp8_ref[...]))
    ve = ln(_dot(x, wve_ref[...]), 2, 3)
    av = _dot(ve, wrvt_ref[...]) + V[9:10, :]
    relv = ln(av + gbv_ref[...], 6, 7)
    p0_ref[...] = relv
    p1_ref[...] = qkb * relv


def _final_body(s0_ref, s1_ref, se_ref, wz_ref, m_ref, rep_ref, vecs_ref,
                o_ref):
    M = m_ref[...]
    V = vecs_ref
    lse = jnp.log(jnp.maximum(se_ref[...], 1e-30))
    lse_b = _dot(lse, rep_ref[...])
    sf = s1_ref[...] - lse_b * s0_ref[...]
    z = jnp.maximum(_dot(sf, wz_ref[...]) + V[2:3, :], 0.0)
    mu = _dot(z, M)
    ex2 = _dot(z * z, M)
    r = lax.rsqrt(ex2 - mu * mu + 1e-5)
    o_ref[...] = (z - mu) * r * V[0:1, :] + V[1:2, :]


_MESH = plsc.VectorSubcoreMesh(core_axis_name="c", subcore_axis_name="s")


@functools.partial(
    pl.kernel,
    mesh=_MESH,
    out_type=[jax.ShapeDtypeStruct((_E, 128), jnp.float32)] * 3,
    scratch_types=[
        pltpu.VMEM((_C,), jnp.int32),
        pltpu.VMEM((_C, 128), jnp.float32),
        pltpu.VMEM((_C, 128), jnp.float32),
        pltpu.SemaphoreType.DMA,
    ],
)
def _gather3(q4_hbm, bk_hbm, bv_hbm, src_hbm, dst_hbm,
             gq_hbm, gbk_hbm, gbv_hbm, idx_v, rows_a, rows_b, sem):
    wid = lax.axis_index("s") * _NC + lax.axis_index("c")
    per = _E // (_NC * _NS)

    def body(i, carry):
        base = wid * per + i * _C
        pltpu.sync_copy(dst_hbm.at[pl.ds(base, _C)], idx_v)
        pltpu.async_copy(q4_hbm.at[idx_v], rows_a, sem).wait()
        pltpu.sync_copy(rows_a, gq_hbm.at[pl.ds(base, _C)])
        pltpu.sync_copy(src_hbm.at[pl.ds(base, _C)], idx_v)
        pltpu.async_copy(bk_hbm.at[idx_v], rows_a, sem).wait()
        pltpu.sync_copy(rows_a, gbk_hbm.at[pl.ds(base, _C)])
        pltpu.async_copy(bv_hbm.at[idx_v], rows_b, sem).wait()
        pltpu.sync_copy(rows_b, gbv_hbm.at[pl.ds(base, _C)])
        return carry

    lax.fori_loop(0, per // _C, body, 0)


_RPT = _N // _NS


@functools.partial(
    pl.kernel,
    mesh=_MESH,
    out_type=[
        jax.ShapeDtypeStruct((_N, 128), jnp.float32),
        jax.ShapeDtypeStruct((_N, 128), jnp.float32),
        jax.ShapeDtypeStruct((_N, 16), jnp.float32),
    ],
    scratch_types=[
        pltpu.VMEM((_C,), jnp.int32),
        pltpu.VMEM((_C, 64), jnp.float32),
        pltpu.VMEM((_C, 64), jnp.float32),
        pltpu.VMEM((_C, 16), jnp.float32),
        pltpu.VMEM((_RPT, 64), jnp.float32),
        pltpu.VMEM((_RPT, 16), jnp.float32),
        pltpu.VMEM_SHARED((_N, 64), jnp.float32),
        pltpu.VMEM_SHARED((_N, 64), jnp.float32),
        pltpu.VMEM_SHARED((_N, 16), jnp.float32),
    ],
)
def _scatter3(p0_hbm, p1_hbm, eq_hbm, dst_hbm, s0_hbm, s1_hbm, se_hbm,
              idx_v, p0buf, p1buf, eqbuf, zbuf, zbuf16, acc0, acc1, acce):
    c = lax.axis_index("c")
    s = lax.axis_index("s")
    row0 = s * _RPT

    def zrow(i, carry):
        for j in range(4):
            zbuf[i, pl.ds(j * 16, 16)] = jnp.zeros((16,), jnp.float32)
        zbuf16[i, pl.ds(0, 16)] = jnp.zeros((16,), jnp.float32)
        return carry

    lax.fori_loop(0, _RPT, zrow, 0)
    pltpu.sync_copy(zbuf, acc0.at[pl.ds(row0, _RPT), :])
    pltpu.sync_copy(zbuf, acc1.at[pl.ds(row0, _RPT), :])
    pltpu.sync_copy(zbuf16, acce.at[pl.ds(row0, _RPT), :])
    plsc.subcore_barrier()

    per = _E // _NS

    def body(i, carry):
        b = s * per + i * _C
        pltpu.sync_copy(dst_hbm.at[pl.ds(b, _C)], idx_v)
        pltpu.sync_copy(p0_hbm.at[pl.ds(b, _C), pl.ds(c * 64, 64)], p0buf)
        pltpu.sync_copy(p1_hbm.at[pl.ds(b, _C), pl.ds(c * 64, 64)], p1buf)
        pltpu.sync_copy(p0buf, acc0.at[idx_v], add=True)
        pltpu.sync_copy(p1buf, acc1.at[idx_v], add=True)

        @pl.when(c == 0)
        def _():
            pltpu.sync_copy(eq_hbm.at[pl.ds(b, _C)], eqbuf)
            pltpu.sync_copy(eqbuf, acce.at[idx_v], add=True)

        return carry

    lax.fori_loop(0, per // _C, body, 0)
    plsc.subcore_barrier()

    pltpu.sync_copy(acc0.at[pl.ds(row0, _RPT), :], zbuf)
    pltpu.sync_copy(zbuf, s0_hbm.at[pl.ds(row0, _RPT), pl.ds(c * 64, 64)])
    pltpu.sync_copy(acc1.at[pl.ds(row0, _RPT), :], zbuf)
    pltpu.sync_copy(zbuf, s1_hbm.at[pl.ds(row0, _RPT), pl.ds(c * 64, 64)])

    @pl.when(c == 0)
    def _():
        pltpu.sync_copy(acce.at[pl.ds(row0, _RPT), :], zbuf16)
        pltpu.sync_copy(zbuf16, se_hbm.at[pl.ds(row0, _RPT), :])


def _block_diag(blk, h):
    return jnp.kron(jnp.eye(h, dtype=jnp.float32), blk.astype(jnp.float32))


def kernel(x_node, x_edge, edge_index, Wq, gq, bq, Wk, gk, bk, Wv, gv, bv,
           Wke, gke, bke, Wve, gve, bve, Wrk, brk, grk_ln, brk_ln,
           Wrv, brv, grv_ln, brv_ln, Wz, bz, gz_ln, bz_ln):
    f32 = jnp.float32
    src = edge_index[0]
    dst = edge_index[1]

    def heads_mat(w):
        return jnp.transpose(w, (1, 0, 2)).reshape(_F, _H * _D).astype(f32)

    M = jnp.asarray(np.kron(np.eye(_H, dtype=np.float32),
                            np.ones((_D, _D), np.float32) / _D))
    Ssum = jnp.asarray(np.kron(np.eye(_H, dtype=np.float32),
                               np.ones((_D, _D), np.float32)))
    sp8 = np.zeros((128, 16), np.float32)
    for h in range(_H):
        sp8[h * _D:(h + 1) * _D, h] = 1.0
    Sp8 = jnp.asarray(sp8)
    rep = np.zeros((16, 128), np.float32)
    for h in range(_H):
        rep[h, h * _D:(h + 1) * _D] = 1.0
    Rep = jnp.asarray(rep)

    nvecs = jnp.stack([gq.reshape(-1), bq.reshape(-1), gk.reshape(-1),
                       bk.reshape(-1), gv.reshape(-1), bv.reshape(-1),
                       jnp.zeros(128, f32), jnp.zeros(128, f32)]).astype(f32)

    evecs = jnp.stack([gke.reshape(-1), bke.reshape(-1),
                       gve.reshape(-1), bve.reshape(-1),
                       jnp.tile(grk_ln, _H), jnp.tile(brk_ln, _H),
                       jnp.tile(grv_ln, _H), jnp.tile(brv_ln, _H),
                       jnp.tile(brk, _H), jnp.tile(brv, _H)]
                      + [jnp.zeros(128, f32)] * 6).astype(f32)

    fvecs = jnp.stack([jnp.tile(gz_ln, _H), jnp.tile(bz_ln, _H),
                       jnp.tile(bz, _H)]
                      + [jnp.zeros(128, f32)] * 5).astype(f32)

    wmat = pl.BlockSpec((128, 128), lambda i: (0, 0))
    vspec8 = pl.BlockSpec((8, 128), lambda i: (0, 0))
    vspec16 = pl.BlockSpec((16, 128), lambda i: (0, 0))

    nb = _N // _BN
    nblk = pl.BlockSpec((_BN, 128), lambda i: (i, 0))
    q4, bkn, bvn = pl.pallas_call(
        _node_body,
        grid=(nb,),
        in_specs=[nblk, wmat, wmat, wmat, wmat, wmat, wmat, vspec8],
        out_specs=[nblk, nblk, nblk],
        out_shape=[jax.ShapeDtypeStruct((_N, 128), f32)] * 3,
    )(x_node.astype(f32), heads_mat(Wq), heads_mat(Wk), heads_mat(Wv),
      _block_diag(Wrk[_D:], _H), _block_diag(Wrv[_D:], _H), M, nvecs)

    gqv, gbk, gbv = _gather3(q4, bkn, bvn, src, dst)

    eb = _E // _BE
    eblk = pl.BlockSpec((_BE, 128), lambda i: (i, 0))
    eblk16 = pl.BlockSpec((_BE, 16), lambda i: (i, 0))
    sp8spec = pl.BlockSpec((128, 16), lambda i: (0, 0))
    p0, p1, eq = pl.pallas_call(
        _edge_body,
        grid=(eb,),
        in_specs=[eblk, eblk, eblk, eblk, wmat, wmat, wmat, wmat, wmat,
                  wmat, sp8spec, vspec16],
        out_specs=[eblk, eblk, eblk16],
        out_shape=[jax.ShapeDtypeStruct((_E, 128), f32),
                   jax.ShapeDtypeStruct((_E, 128), f32),
                   jax.ShapeDtypeStruct((_E, 16), f32)],
    )(x_edge.astype(f32), gqv, gbk, gbv, heads_mat(Wke), heads_mat(Wve),
      _block_diag(Wrk[:_D], _H), _block_diag(Wrv[:_D], _H), M, Ssum, Sp8,
      evecs)

    s0, s1, sexp = _scatter3(p0, p1, eq, dst)

    repspec = pl.BlockSpec((16, 128), lambda i: (0, 0))
    seblk = pl.BlockSpec((_BN, 16), lambda i: (i, 0))
    out = pl.pallas_call(
        _final_body,
        grid=(nb,),
        in_specs=[nblk, nblk, seblk, wmat, wmat, repspec, vspec8],
        out_specs=nblk,
        out_shape=jax.ShapeDtypeStruct((_N, 128), f32),
    )(s0, s1, sexp, _block_diag(Wz, _H), M, Rep, fvecs)

    return out


# speedup vs baseline: 20.6312x; 20.6312x over previous
import functools

import jax
import jax.numpy as jnp
import numpy as np
from jax import lax
from jax.experimental import pallas as pl
from jax.experimental.pallas import tpu as pltpu
from jax.experimental.pallas import tpu_sc as plsc

_N = 10000
_E = 320000
_F = 128
_H = 8
_D = 16

_NC = 2
_NS = 16
_C = 80

_BN = 400
_BE = 512


def _dot(a, b):
    return jnp.dot(a, b, preferred_element_type=jnp.float32)


def _node_body(x_ref, wq_ref, wk_ref, wv_ref, wrkb_ref, wrvb_ref, m_ref,
               vecs_ref, q4_ref, bk_ref, bv_ref):
    x = x_ref[...]
    M = m_ref[...]
    V = vecs_ref

    def ln(y, gi, bi):
        mu = _dot(y, M)
        ex2 = _dot(y * y, M)
        r = lax.rsqrt(ex2 - mu * mu + 1e-5)
        return (y - mu) * r * V[gi:gi + 1, :] + V[bi:bi + 1, :]

    q = ln(_dot(x, wq_ref[...]), 0, 1)
    q4_ref[...] = q * 0.25
    k = ln(_dot(x, wk_ref[...]), 2, 3)
    bk_ref[...] = _dot(k, wrkb_ref[...])
    v = ln(_dot(x, wv_ref[...]), 4, 5)
    bv_ref[...] = _dot(v, wrvb_ref[...])


def _edge_body(xe_ref, gq_ref, gbk_ref, gbv_ref, wke_ref, wve_ref, wrkt_ref,
               wrvt_ref, m_ref, ssum_ref, vecs_ref,
               p0_ref, p1_ref, eq_ref):
    x = xe_ref[...]
    M = m_ref[...]
    V = vecs_ref

    def ln(y, gi, bi):
        mu = _dot(y, M)
        ex2 = _dot(y * y, M)
        r = lax.rsqrt(ex2 - mu * mu + 1e-5)
        return (y - mu) * r * V[gi:gi + 1, :] + V[bi:bi + 1, :]

    ke = ln(_dot(x, wke_ref[...]), 0, 1)
    ak = _dot(ke, wrkt_ref[...]) + V[8:9, :]
    relk = ln(ak + gbk_ref[...], 4, 5)
    prod = gq_ref[...] * relk
    qkb = _dot(prod, ssum_ref[...])
    eq_ref[...] = jnp.exp(qkb)
    ve = ln(_dot(x, wve_ref[...]), 2, 3)
    av = _dot(ve, wrvt_ref[...]) + V[9:10, :]
    relv = ln(av + gbv_ref[...], 6, 7)
    p0_ref[...] = relv
    p1_ref[...] = qkb * relv


def _final_body(s0_ref, s1_ref, se_ref, wz_ref, m_ref, vecs_ref, o_ref):
    M = m_ref[...]
    V = vecs_ref
    lse_b = jnp.log(jnp.maximum(se_ref[...], 1e-30))
    sf = s1_ref[...] - lse_b * s0_ref[...]
    z = jnp.maximum(_dot(sf, wz_ref[...]) + V[2:3, :], 0.0)
    mu = _dot(z, M)
    ex2 = _dot(z * z, M)
    r = lax.rsqrt(ex2 - mu * mu + 1e-5)
    o_ref[...] = (z - mu) * r * V[0:1, :] + V[1:2, :]


@functools.cache
def _sc_kernels():
    mesh = plsc.VectorSubcoreMesh(core_axis_name="c", subcore_axis_name="s")

    @functools.partial(
        pl.kernel,
        mesh=mesh,
        out_type=[jax.ShapeDtypeStruct((_E, 128), jnp.float32)] * 3,
        scratch_types=[
            pltpu.VMEM((_C,), jnp.int32),
            pltpu.VMEM((_C, 128), jnp.float32),
            pltpu.VMEM((_C, 128), jnp.float32),
            pltpu.SemaphoreType.DMA,
        ],
    )
    def _gather3(q4_hbm, bk_hbm, bv_hbm, src_hbm, dst_hbm,
                 gq_hbm, gbk_hbm, gbv_hbm, idx_v, rows_a, rows_b, sem):
        wid = lax.axis_index("s") * _NC + lax.axis_index("c")
        per = _E // (_NC * _NS)

        def body(i, carry):
            base = wid * per + i * _C
            pltpu.sync_copy(dst_hbm.at[pl.ds(base, _C)], idx_v)
            pltpu.async_copy(q4_hbm.at[idx_v], rows_a, sem).wait()
            pltpu.sync_copy(rows_a, gq_hbm.at[pl.ds(base, _C)])
            pltpu.sync_copy(src_hbm.at[pl.ds(base, _C)], idx_v)
            pltpu.async_copy(bk_hbm.at[idx_v], rows_a, sem).wait()
            pltpu.sync_copy(rows_a, gbk_hbm.at[pl.ds(base, _C)])
            pltpu.async_copy(bv_hbm.at[idx_v], rows_b, sem).wait()
            pltpu.sync_copy(rows_b, gbv_hbm.at[pl.ds(base, _C)])
            return carry

        lax.fori_loop(0, per // _C, body, 0)

    @functools.partial(
        pl.kernel,
        mesh=mesh,
        out_type=[
            jax.ShapeDtypeStruct((_NP, 128), jnp.float32),
            jax.ShapeDtypeStruct((_NP, 128), jnp.float32),
        ],
        scratch_types=[
            pltpu.VMEM((_C,), jnp.int32),
            pltpu.VMEM((_C,), jnp.int32),
            pltpu.VMEM((_C, 128), jnp.float32),
            pltpu.VMEM_SHARED((_NP, 128), jnp.float32),
            pltpu.SemaphoreType.DMA,
        ],
    )
    def _scatter2(p0_hbm, p1_hbm, dst_hbm, s0_hbm, s1_hbm,
                  idx_v, idx_r, pbuf, acc, sem):
        c = lax.axis_index("c")
        s = lax.axis_index("s")
        row0 = s * _RPT

        def set_idx_r(base):
            for j in range(_C // 16):
                idx_r[pl.ds(j * 16, 16)] = base + j * 16 + lax.iota(
                    jnp.int32, 16)

        def zrow(i, carry):
            for j in range(8):
                pbuf[i, pl.ds(j * 16, 16)] = jnp.zeros((16,), jnp.float32)
            return carry

        lax.fori_loop(0, _C, zrow, 0)

        def zacc(i, carry):
            set_idx_r(row0 + i * _C)
            pltpu.sync_copy(pbuf, acc.at[idx_r])
            return carry

        lax.fori_loop(0, _RPT // _C, zacc, 0)
        plsc.subcore_barrier()

        per = _E // _NS

        def body(i, carry):
            b = s * per + i * _C
            pltpu.sync_copy(dst_hbm.at[pl.ds(b, _C)], idx_v)

            @pl.when(c == 0)
            def _():
                pltpu.sync_copy(p0_hbm.at[pl.ds(b, _C), :], pbuf)

            @pl.when(c == 1)
            def _():
                pltpu.sync_copy(p1_hbm.at[pl.ds(b, _C), :], pbuf)

            pltpu.sync_copy(pbuf, acc.at[idx_v], add=True)
            return carry

        lax.fori_loop(0, per // _C, body, 0)
        plsc.subcore_barrier()

        def wb(i, carry):
            r = row0 + i * _C
            set_idx_r(r)
            pltpu.async_copy(acc.at[idx_r], pbuf, sem).wait()

            @pl.when(c == 0)
            def _():
                pltpu.sync_copy(pbuf, s0_hbm.at[pl.ds(r, _C), :])

            @pl.when(c == 1)
            def _():
                pltpu.sync_copy(pbuf, s1_hbm.at[pl.ds(r, _C), :])

            return carry

        lax.fori_loop(0, _RPT // _C, wb, 0)

    mesh1 = plsc.VectorSubcoreMesh(core_axis_name="c", subcore_axis_name="s",
                                   num_cores=1)

    @functools.partial(
        pl.kernel,
        mesh=mesh1,
        out_type=jax.ShapeDtypeStruct((_NP, 128), jnp.float32),
        scratch_types=[
            pltpu.VMEM((_C,), jnp.int32),
            pltpu.VMEM((_C,), jnp.int32),
            pltpu.VMEM((_C, 128), jnp.float32),
            pltpu.VMEM_SHARED((_NP, 128), jnp.float32),
            pltpu.SemaphoreType.DMA,
        ],
    )
    def _scatter_e(eq_hbm, dst_hbm, se_hbm, idx_v, idx_r, ebuf, acc, sem):
        s = lax.axis_index("s")
        row0 = s * _RPT

        def set_idx_r(base):
            for j in range(_C // 16):
                idx_r[pl.ds(j * 16, 16)] = base + j * 16 + lax.iota(
                    jnp.int32, 16)

        def zrow(i, carry):
            for j in range(8):
                ebuf[i, pl.ds(j * 16, 16)] = jnp.zeros((16,), jnp.float32)
            return carry

        lax.fori_loop(0, _C, zrow, 0)

        def zacc(i, carry):
            set_idx_r(row0 + i * _C)
            pltpu.sync_copy(ebuf, acc.at[idx_r])
            return carry

        lax.fori_loop(0, _RPT // _C, zacc, 0)
        plsc.subcore_barrier()

        per = _E // _NS

        def body(i, carry):
            b = s * per + i * _C
            pltpu.sync_copy(dst_hbm.at[pl.ds(b, _C)], idx_v)
            pltpu.sync_copy(eq_hbm.at[pl.ds(b, _C), :], ebuf)
            pltpu.sync_copy(ebuf, acc.at[idx_v], add=True)
            return carry

        lax.fori_loop(0, per // _C, body, 0)
        plsc.subcore_barrier()

        def wb(i, carry):
            r = row0 + i * _C
            set_idx_r(r)
            pltpu.async_copy(acc.at[idx_r], ebuf, sem).wait()
            pltpu.sync_copy(ebuf, se_hbm.at[pl.ds(r, _C), :])
            return carry

        lax.fori_loop(0, _RPT // _C, wb, 0)

    return _gather3, _scatter2, _scatter_e


_NP = 10240
_RPT = _NP // _NS


def _block_diag(blk, h):
    return jnp.kron(jnp.eye(h, dtype=jnp.float32), blk.astype(jnp.float32))


def kernel(x_node, x_edge, edge_index, Wq, gq, bq, Wk, gk, bk, Wv, gv, bv,
           Wke, gke, bke, Wve, gve, bve, Wrk, brk, grk_ln, brk_ln,
           Wrv, brv, grv_ln, brv_ln, Wz, bz, gz_ln, bz_ln):
    f32 = jnp.float32
    src = edge_index[0]
    dst = edge_index[1]

    def heads_mat(w):
        return jnp.transpose(w, (1, 0, 2)).reshape(_F, _H * _D).astype(f32)

    M = jnp.asarray(np.kron(np.eye(_H, dtype=np.float32),
                            np.ones((_D, _D), np.float32) / _D))
    Ssum = jnp.asarray(np.kron(np.eye(_H, dtype=np.float32),
                               np.ones((_D, _D), np.float32)))
    sp8 = np.zeros((128, 16), np.float32)
    for h in range(_H):
        sp8[h * _D:(h + 1) * _D, h] = 1.0
    Sp8 = jnp.asarray(sp8)
    rep = np.zeros((16, 128), np.float32)
    for h in range(_H):
        rep[h, h * _D:(h + 1) * _D] = 1.0
    Rep = jnp.asarray(rep)

    nvecs = jnp.stack([gq.reshape(-1), bq.reshape(-1), gk.reshape(-1),
                       bk.reshape(-1), gv.reshape(-1), bv.reshape(-1),
                       jnp.zeros(128, f32), jnp.zeros(128, f32)]).astype(f32)

    evecs = jnp.stack([gke.reshape(-1), bke.reshape(-1),
                       gve.reshape(-1), bve.reshape(-1),
                       jnp.tile(grk_ln, _H), jnp.tile(brk_ln, _H),
                       jnp.tile(grv_ln, _H), jnp.tile(brv_ln, _H),
                       jnp.tile(brk, _H), jnp.tile(brv, _H)]
                      + [jnp.zeros(128, f32)] * 6).astype(f32)

    fvecs = jnp.stack([jnp.tile(gz_ln, _H), jnp.tile(bz_ln, _H),
                       jnp.tile(bz, _H)]
                      + [jnp.zeros(128, f32)] * 5).astype(f32)

    wmat = pl.BlockSpec((128, 128), lambda i: (0, 0))
    vspec8 = pl.BlockSpec((8, 128), lambda i: (0, 0))
    vspec16 = pl.BlockSpec((16, 128), lambda i: (0, 0))

    nb = _N // _BN
    nblk = pl.BlockSpec((_BN, 128), lambda i: (i, 0))
    q4, bkn, bvn = pl.pallas_call(
        _node_body,
        grid=(nb,),
        in_specs=[nblk, wmat, wmat, wmat, wmat, wmat, wmat, vspec8],
        out_specs=[nblk, nblk, nblk],
        out_shape=[jax.ShapeDtypeStruct((_N, 128), f32)] * 3,
    )(x_node.astype(f32), heads_mat(Wq), heads_mat(Wk), heads_mat(Wv),
      _block_diag(Wrk[_D:], _H), _block_diag(Wrv[_D:], _H), M, nvecs)

    gather3, scatter2, scatter_e = _sc_kernels()
    gqv, gbk, gbv = gather3(q4, bkn, bvn, src, dst)

    eb = _E // _BE
    eblk = pl.BlockSpec((_BE, 128), lambda i: (i, 0))
    p0, p1, eq = pl.pallas_call(
        _edge_body,
        grid=(eb,),
        in_specs=[eblk, eblk, eblk, eblk, wmat, wmat, wmat, wmat, wmat,
                  wmat, vspec16],
        out_specs=[eblk, eblk, eblk],
        out_shape=[jax.ShapeDtypeStruct((_E, 128), f32)] * 3,
    )(x_edge.astype(f32), gqv, gbk, gbv, heads_mat(Wke), heads_mat(Wve),
      _block_diag(Wrk[:_D], _H), _block_diag(Wrv[:_D], _H), M, Ssum,
      evecs)

    s0p, s1p = scatter2(p0, p1, dst)
    sep = scatter_e(eq, dst)
    s0, s1, sexp = s0p[:_N], s1p[:_N], sep[:_N]

    out = pl.pallas_call(
        _final_body,
        grid=(nb,),
        in_specs=[nblk, nblk, nblk, wmat, wmat, vspec8],
        out_specs=nblk,
        out_shape=jax.ShapeDtypeStruct((_N, 128), f32),
    )(s0, s1, sexp, _block_diag(Wz, _H), M, fvecs)

    return out


# speedup vs baseline: 29.7631x; 1.4426x over previous
import functools

import jax
import jax.numpy as jnp
import numpy as np
from jax import lax
from jax.experimental import pallas as pl
from jax.experimental.pallas import tpu as pltpu
from jax.experimental.pallas import tpu_sc as plsc

_N = 10000
_E = 320000
_F = 128
_H = 8
_D = 16

_NC = 2
_NS = 16
_C = 80

_BN = 400
_BE = 512


def _dot(a, b):
    return jnp.dot(a, b, preferred_element_type=jnp.float32)


def _node_body(x_ref, wq_ref, wk_ref, wv_ref, wrkb_ref, wrvb_ref, m_ref,
               vecs_ref, q4_ref, bk_ref, bv_ref):
    x = x_ref[...]
    M = m_ref[...]
    V = vecs_ref

    def ln(y, gi, bi):
        mu = _dot(y, M)
        ex2 = _dot(y * y, M)
        r = lax.rsqrt(ex2 - mu * mu + 1e-5)
        return (y - mu) * r * V[gi:gi + 1, :] + V[bi:bi + 1, :]

    q = ln(_dot(x, wq_ref[...]), 0, 1)
    q4_ref[...] = q * 0.25
    k = ln(_dot(x, wk_ref[...]), 2, 3)
    bk_ref[...] = _dot(k, wrkb_ref[...])
    v = ln(_dot(x, wv_ref[...]), 4, 5)
    bv_ref[...] = _dot(v, wrvb_ref[...])


def _edge_body(xe_ref, gq_ref, gbk_ref, gbv_ref, wke_ref, wve_ref, wrkt_ref,
               wrvt_ref, m_ref, ssum_ref, vecs_ref,
               p0_ref, p1_ref, eq_ref):
    x = xe_ref[...]
    M = m_ref[...]
    V = vecs_ref

    def ln(y, gi, bi):
        mu = _dot(y, M)
        ex2 = _dot(y * y, M)
        r = lax.rsqrt(ex2 - mu * mu + 1e-5)
        return (y - mu) * r * V[gi:gi + 1, :] + V[bi:bi + 1, :]

    ke = ln(_dot(x, wke_ref[...]), 0, 1)
    ak = _dot(ke, wrkt_ref[...]) + V[8:9, :]
    relk = ln(ak + gbk_ref[...], 4, 5)
    prod = gq_ref[...] * relk
    qkb = _dot(prod, ssum_ref[...])
    eq_ref[...] = jnp.exp(qkb)
    ve = ln(_dot(x, wve_ref[...]), 2, 3)
    av = _dot(ve, wrvt_ref[...]) + V[9:10, :]
    relv = ln(av + gbv_ref[...], 6, 7)
    p0_ref[...] = relv
    p1_ref[...] = qkb * relv


def _final_body(s0_ref, s1_ref, se_ref, wz_ref, m_ref, vecs_ref, o_ref):
    M = m_ref[...]
    V = vecs_ref
    lse_b = jnp.log(jnp.maximum(se_ref[...], 1e-30))
    sf = s1_ref[...] - lse_b * s0_ref[...]
    z = jnp.maximum(_dot(sf, wz_ref[...]) + V[2:3, :], 0.0)
    mu = _dot(z, M)
    ex2 = _dot(z * z, M)
    r = lax.rsqrt(ex2 - mu * mu + 1e-5)
    o_ref[...] = (z - mu) * r * V[0:1, :] + V[1:2, :]


@functools.cache
def _sc_kernels():
    mesh = plsc.VectorSubcoreMesh(core_axis_name="c", subcore_axis_name="s")

    _GU = 3
    _SU = 4

    @functools.partial(
        pl.kernel,
        mesh=mesh,
        out_type=[jax.ShapeDtypeStruct((_E, 128), jnp.float32)] * 3,
        scratch_types=(
            [pltpu.VMEM((_C,), jnp.int32)] * (2 * _GU)
            + [pltpu.VMEM((_C, 128), jnp.float32)] * (3 * _GU)
            + [pltpu.SemaphoreType.DMA] * 3
        ),
    )
    def _gather3(q4_hbm, bk_hbm, bv_hbm, src_hbm, dst_hbm,
                 gq_hbm, gbk_hbm, gbv_hbm, *scr):
        idx_d = scr[:_GU]
        idx_s = scr[_GU:2 * _GU]
        buf_q = scr[2 * _GU:2 * _GU + _GU]
        buf_k = scr[2 * _GU + _GU:2 * _GU + 2 * _GU]
        buf_v = scr[2 * _GU + 2 * _GU:2 * _GU + 3 * _GU]
        sem_i, sem_g, sem_w = scr[-3:]
        wid = lax.axis_index("s") * _NC + lax.axis_index("c")
        per = _E // (_NC * _NS)
        nfull = (per // _C) // _GU
        ntail = (per // _C) % _GU

        def do_group(first_chunk, nu):
            hs = []
            for u in range(nu):
                b = wid * per + (first_chunk + u) * _C
                hs.append(pltpu.async_copy(dst_hbm.at[pl.ds(b, _C)],
                                           idx_d[u], sem_i))
                hs.append(pltpu.async_copy(src_hbm.at[pl.ds(b, _C)],
                                           idx_s[u], sem_i))
            for h in hs:
                h.wait()
            hs = []
            for u in range(nu):
                hs.append(pltpu.async_copy(q4_hbm.at[idx_d[u]], buf_q[u],
                                           sem_g))
                hs.append(pltpu.async_copy(bk_hbm.at[idx_s[u]], buf_k[u],
                                           sem_g))
                hs.append(pltpu.async_copy(bv_hbm.at[idx_s[u]], buf_v[u],
                                           sem_g))
            for h in hs:
                h.wait()
            hs = []
            for u in range(nu):
                b = wid * per + (first_chunk + u) * _C
                hs.append(pltpu.async_copy(buf_q[u],
                                           gq_hbm.at[pl.ds(b, _C)], sem_w))
                hs.append(pltpu.async_copy(buf_k[u],
                                           gbk_hbm.at[pl.ds(b, _C)], sem_w))
                hs.append(pltpu.async_copy(buf_v[u],
                                           gbv_hbm.at[pl.ds(b, _C)], sem_w))
            for h in hs:
                h.wait()

        def body(i, carry):
            do_group(i * _GU, _GU)
            return carry

        lax.fori_loop(0, nfull, body, 0)
        if ntail:
            do_group(nfull * _GU, ntail)

    @functools.partial(
        pl.kernel,
        mesh=mesh,
        out_type=[
            jax.ShapeDtypeStruct((_NP, 128), jnp.float32),
            jax.ShapeDtypeStruct((_NP, 128), jnp.float32),
        ],
        scratch_types=(
            [pltpu.VMEM((_C,), jnp.int32)] * _SU
            + [pltpu.VMEM((_C, 128), jnp.float32)] * _SU
            + [pltpu.VMEM((_C,), jnp.int32),
               pltpu.VMEM_SHARED((_NP, 128), jnp.float32)]
            + [pltpu.SemaphoreType.DMA] * 2
        ),
    )
    def _scatter2(p0_hbm, p1_hbm, dst_hbm, s0_hbm, s1_hbm, *scr):
        idx_v = scr[:_SU]
        pbuf = scr[_SU:2 * _SU]
        idx_r, acc = scr[2 * _SU], scr[2 * _SU + 1]
        sem_i, sem_s = scr[-2:]
        c = lax.axis_index("c")
        s = lax.axis_index("s")
        row0 = s * _RPT

        def set_idx_r(base):
            for j in range(_C // 16):
                idx_r[pl.ds(j * 16, 16)] = base + j * 16 + lax.iota(
                    jnp.int32, 16)

        def zrow(i, carry):
            for j in range(8):
                pbuf[0][i, pl.ds(j * 16, 16)] = jnp.zeros((16,), jnp.float32)
            return carry

        lax.fori_loop(0, _C, zrow, 0)

        def zacc(i, carry):
            set_idx_r(row0 + i * _C)
            pltpu.sync_copy(pbuf[0], acc.at[idx_r])
            return carry

        lax.fori_loop(0, _RPT // _C, zacc, 0)
        plsc.subcore_barrier()

        per = _E // _NS
        nfull = (per // _C) // _SU
        ntail = (per // _C) % _SU

        def do_group(first_chunk, nu):
            hs = []
            for u in range(nu):
                b = s * per + (first_chunk + u) * _C
                hs.append(pltpu.async_copy(dst_hbm.at[pl.ds(b, _C)],
                                           idx_v[u], sem_i))

            @pl.when(c == 0)
            def _():
                h2 = [pltpu.async_copy(
                    p0_hbm.at[pl.ds(s * per + (first_chunk + u) * _C, _C), :],
                    pbuf[u], sem_s) for u in range(nu)]
                for h in h2:
                    h.wait()

            @pl.when(c == 1)
            def _():
                h2 = [pltpu.async_copy(
                    p1_hbm.at[pl.ds(s * per + (first_chunk + u) * _C, _C), :],
                    pbuf[u], sem_s) for u in range(nu)]
                for h in h2:
                    h.wait()

            for h in hs:
                h.wait()
            hs = []
            for u in range(nu):
                hs.append(pltpu.async_copy(pbuf[u], acc.at[idx_v[u]], sem_s,
                                           add=True))
            for h in hs:
                h.wait()

        def body(i, carry):
            do_group(i * _SU, _SU)
            return carry

        lax.fori_loop(0, nfull, body, 0)
        if ntail:
            do_group(nfull * _SU, ntail)
        plsc.subcore_barrier()

        def wb(i, carry):
            r = row0 + i * _C
            set_idx_r(r)
            pltpu.async_copy(acc.at[idx_r], pbuf[0], sem_i).wait()

            @pl.when(c == 0)
            def _():
                pltpu.sync_copy(pbuf[0], s0_hbm.at[pl.ds(r, _C), :])

            @pl.when(c == 1)
            def _():
                pltpu.sync_copy(pbuf[0], s1_hbm.at[pl.ds(r, _C), :])

            return carry

        lax.fori_loop(0, _RPT // _C, wb, 0)

    mesh1 = plsc.VectorSubcoreMesh(core_axis_name="c", subcore_axis_name="s",
                                   num_cores=1)

    @functools.partial(
        pl.kernel,
        mesh=mesh1,
        out_type=jax.ShapeDtypeStruct((_NP, 128), jnp.float32),
        scratch_types=(
            [pltpu.VMEM((_C,), jnp.int32)] * _SU
            + [pltpu.VMEM((_C, 128), jnp.float32)] * _SU
            + [pltpu.VMEM((_C,), jnp.int32),
               pltpu.VMEM_SHARED((_NP, 128), jnp.float32)]
            + [pltpu.SemaphoreType.DMA] * 2
        ),
    )
    def _scatter_e(eq_hbm, dst_hbm, se_hbm, *scr):
        idx_v = scr[:_SU]
        ebuf = scr[_SU:2 * _SU]
        idx_r, acc = scr[2 * _SU], scr[2 * _SU + 1]
        sem_i, sem_s = scr[-2:]
        s = lax.axis_index("s")
        row0 = s * _RPT

        def set_idx_r(base):
            for j in range(_C // 16):
                idx_r[pl.ds(j * 16, 16)] = base + j * 16 + lax.iota(
                    jnp.int32, 16)

        def zrow(i, carry):
            for j in range(8):
                ebuf[0][i, pl.ds(j * 16, 16)] = jnp.zeros((16,), jnp.float32)
            return carry

        lax.fori_loop(0, _C, zrow, 0)

        def zacc(i, carry):
            set_idx_r(row0 + i * _C)
            pltpu.sync_copy(ebuf[0], acc.at[idx_r])
            return carry

        lax.fori_loop(0, _RPT // _C, zacc, 0)
        plsc.subcore_barrier()

        per = _E // _NS
        nfull = (per // _C) // _SU
        ntail = (per // _C) % _SU

        def do_group(first_chunk, nu):
            hs = []
            for u in range(nu):
                b = s * per + (first_chunk + u) * _C
                hs.append(pltpu.async_copy(dst_hbm.at[pl.ds(b, _C)],
                                           idx_v[u], sem_i))
                hs.append(pltpu.async_copy(eq_hbm.at[pl.ds(b, _C), :],
                                           ebuf[u], sem_s))
            for h in hs:
                h.wait()
            hs = []
            for u in range(nu):
                hs.append(pltpu.async_copy(ebuf[u], acc.at[idx_v[u]], sem_s,
                                           add=True))
            for h in hs:
                h.wait()

        def body(i, carry):
            do_group(i * _SU, _SU)
            return carry

        lax.fori_loop(0, nfull, body, 0)
        if ntail:
            do_group(nfull * _SU, ntail)
        plsc.subcore_barrier()

        def wb(i, carry):
            r = row0 + i * _C
            set_idx_r(r)
            pltpu.async_copy(acc.at[idx_r], ebuf[0], sem_i).wait()
            pltpu.sync_copy(ebuf[0], se_hbm.at[pl.ds(r, _C), :])
            return carry

        lax.fori_loop(0, _RPT // _C, wb, 0)

    return _gather3, _scatter2, _scatter_e


_NP = 10240
_RPT = _NP // _NS


def _block_diag(blk, h):
    return jnp.kron(jnp.eye(h, dtype=jnp.float32), blk.astype(jnp.float32))


def kernel(x_node, x_edge, edge_index, Wq, gq, bq, Wk, gk, bk, Wv, gv, bv,
           Wke, gke, bke, Wve, gve, bve, Wrk, brk, grk_ln, brk_ln,
           Wrv, brv, grv_ln, brv_ln, Wz, bz, gz_ln, bz_ln):
    f32 = jnp.float32
    src = edge_index[0]
    dst = edge_index[1]

    def heads_mat(w):
        return jnp.transpose(w, (1, 0, 2)).reshape(_F, _H * _D).astype(f32)

    M = jnp.asarray(np.kron(np.eye(_H, dtype=np.float32),
                            np.ones((_D, _D), np.float32) / _D))
    Ssum = jnp.asarray(np.kron(np.eye(_H, dtype=np.float32),
                               np.ones((_D, _D), np.float32)))
    sp8 = np.zeros((128, 16), np.float32)
    for h in range(_H):
        sp8[h * _D:(h + 1) * _D, h] = 1.0
    Sp8 = jnp.asarray(sp8)
    rep = np.zeros((16, 128), np.float32)
    for h in range(_H):
        rep[h, h * _D:(h + 1) * _D] = 1.0
    Rep = jnp.asarray(rep)

    nvecs = jnp.stack([gq.reshape(-1), bq.reshape(-1), gk.reshape(-1),
                       bk.reshape(-1), gv.reshape(-1), bv.reshape(-1),
                       jnp.zeros(128, f32), jnp.zeros(128, f32)]).astype(f32)

    evecs = jnp.stack([gke.reshape(-1), bke.reshape(-1),
                       gve.reshape(-1), bve.reshape(-1),
                       jnp.tile(grk_ln, _H), jnp.tile(brk_ln, _H),
                       jnp.tile(grv_ln, _H), jnp.tile(brv_ln, _H),
                       jnp.tile(brk, _H), jnp.tile(brv, _H)]
                      + [jnp.zeros(128, f32)] * 6).astype(f32)

    fvecs = jnp.stack([jnp.tile(gz_ln, _H), jnp.tile(bz_ln, _H),
                       jnp.tile(bz, _H)]
                      + [jnp.zeros(128, f32)] * 5).astype(f32)

    wmat = pl.BlockSpec((128, 128), lambda i: (0, 0))
    vspec8 = pl.BlockSpec((8, 128), lambda i: (0, 0))
    vspec16 = pl.BlockSpec((16, 128), lambda i: (0, 0))

    nb = _N // _BN
    nblk = pl.BlockSpec((_BN, 128), lambda i: (i, 0))
    q4, bkn, bvn = pl.pallas_call(
        _node_body,
        grid=(nb,),
        in_specs=[nblk, wmat, wmat, wmat, wmat, wmat, wmat, vspec8],
        out_specs=[nblk, nblk, nblk],
        out_shape=[jax.ShapeDtypeStruct((_N, 128), f32)] * 3,
    )(x_node.astype(f32), heads_mat(Wq), heads_mat(Wk), heads_mat(Wv),
      _block_diag(Wrk[_D:], _H), _block_diag(Wrv[_D:], _H), M, nvecs)

    gather3, scatter2, scatter_e = _sc_kernels()
    gqv, gbk, gbv = gather3(q4, bkn, bvn, src, dst)

    eb = _E // _BE
    eblk = pl.BlockSpec((_BE, 128), lambda i: (i, 0))
    p0, p1, eq = pl.pallas_call(
        _edge_body,
        grid=(eb,),
        in_specs=[eblk, eblk, eblk, eblk, wmat, wmat, wmat, wmat, wmat,
                  wmat, vspec16],
        out_specs=[eblk, eblk, eblk],
        out_shape=[jax.ShapeDtypeStruct((_E, 128), f32)] * 3,
    )(x_edge.astype(f32), gqv, gbk, gbv, heads_mat(Wke), heads_mat(Wve),
      _block_diag(Wrk[:_D], _H), _block_diag(Wrv[:_D], _H), M, Ssum,
      evecs)

    s0p, s1p = scatter2(p0, p1, dst)
    sep = scatter_e(eq, dst)
    s0, s1, sexp = s0p[:_N], s1p[:_N], sep[:_N]

    out = pl.pallas_call(
        _final_body,
        grid=(nb,),
        in_specs=[nblk, nblk, nblk, wmat, wmat, vspec8],
        out_specs=nblk,
        out_shape=jax.ShapeDtypeStruct((_N, 128), f32),
    )(s0, s1, sexp, _block_diag(Wz, _H), M, fvecs)

    return out


# speedup vs baseline: 29.9268x; 1.0055x over previous
import functools

import jax
import jax.numpy as jnp
import numpy as np
from jax import lax
from jax.experimental import pallas as pl
from jax.experimental.pallas import tpu as pltpu
from jax.experimental.pallas import tpu_sc as plsc

_N = 10000
_E = 320000
_F = 128
_H = 8
_D = 16

_NC = 2
_NS = 16
_C = 80

_BN = 400
_BE = 512


def _dot(a, b):
    return jnp.dot(a, b, preferred_element_type=jnp.float32)


def _node_body(x_ref, wq_ref, wk_ref, wv_ref, wrkb_ref, wrvb_ref, m_ref,
               vecs_ref, q4_ref, bk_ref, bv_ref):
    x = x_ref[...]
    M = m_ref[...]
    V = vecs_ref

    def ln(y, gi, bi):
        mu = _dot(y, M)
        ex2 = _dot(y * y, M)
        r = lax.rsqrt(ex2 - mu * mu + 1e-5)
        return (y - mu) * r * V[gi:gi + 1, :] + V[bi:bi + 1, :]

    q = ln(_dot(x, wq_ref[...]), 0, 1)
    q4_ref[...] = q * 0.25
    k = ln(_dot(x, wk_ref[...]), 2, 3)
    bk_ref[...] = _dot(k, wrkb_ref[...])
    v = ln(_dot(x, wv_ref[...]), 4, 5)
    bv_ref[...] = _dot(v, wrvb_ref[...])


def _edge_body(xe_ref, gq_ref, gbk_ref, gbv_ref, wke_ref, wve_ref, wrkt_ref,
               wrvt_ref, m_ref, ssum_ref, vecs_ref,
               p0_ref, p1_ref, eq_ref):
    x = xe_ref[...]
    M = m_ref[...]
    V = vecs_ref

    def ln(y, gi, bi):
        mu = _dot(y, M)
        ex2 = _dot(y * y, M)
        r = lax.rsqrt(ex2 - mu * mu + 1e-5)
        return (y - mu) * r * V[gi:gi + 1, :] + V[bi:bi + 1, :]

    ke = ln(_dot(x, wke_ref[...]), 0, 1)
    ak = _dot(ke, wrkt_ref[...]) + V[8:9, :]
    relk = ln(ak + gbk_ref[...], 4, 5)
    prod = gq_ref[...] * relk
    qkb = _dot(prod, ssum_ref[...])
    eq_ref[...] = jnp.exp(qkb)
    ve = ln(_dot(x, wve_ref[...]), 2, 3)
    av = _dot(ve, wrvt_ref[...]) + V[9:10, :]
    relv = ln(av + gbv_ref[...], 6, 7)
    p0_ref[...] = relv
    p1_ref[...] = qkb * relv


def _final_body(s0_ref, s1_ref, se_ref, wz_ref, m_ref, vecs_ref, o_ref):
    M = m_ref[...]
    V = vecs_ref
    lse_b = jnp.log(jnp.maximum(se_ref[...], 1e-30))
    sf = s1_ref[...] - lse_b * s0_ref[...]
    z = jnp.maximum(_dot(sf, wz_ref[...]) + V[2:3, :], 0.0)
    mu = _dot(z, M)
    ex2 = _dot(z * z, M)
    r = lax.rsqrt(ex2 - mu * mu + 1e-5)
    o_ref[...] = (z - mu) * r * V[0:1, :] + V[1:2, :]


@functools.cache
def _sc_kernels():
    mesh = plsc.VectorSubcoreMesh(core_axis_name="c", subcore_axis_name="s")

    _GU = 4
    _SU = 4

    @functools.partial(
        pl.kernel,
        mesh=mesh,
        out_type=[jax.ShapeDtypeStruct((_E, 128), jnp.float32)] * 3,
        scratch_types=(
            [pltpu.VMEM((_C,), jnp.int32)] * (2 * _GU)
            + [pltpu.VMEM((_C, 128), jnp.float32)] * (3 * _GU)
            + [pltpu.SemaphoreType.DMA] * 3
        ),
    )
    def _gather3(q4_hbm, bk_hbm, bv_hbm, src_hbm, dst_hbm,
                 gq_hbm, gbk_hbm, gbv_hbm, *scr):
        idx_d = scr[:_GU]
        idx_s = scr[_GU:2 * _GU]
        buf_q = scr[2 * _GU:2 * _GU + _GU]
        buf_k = scr[2 * _GU + _GU:2 * _GU + 2 * _GU]
        buf_v = scr[2 * _GU + 2 * _GU:2 * _GU + 3 * _GU]
        sem_i, sem_g, sem_w = scr[-3:]
        wid = lax.axis_index("s") * _NC + lax.axis_index("c")
        per = _E // (_NC * _NS)
        nfull = (per // _C) // _GU
        ntail = (per // _C) % _GU

        def do_group(first_chunk, nu):
            hs = []
            for u in range(nu):
                b = wid * per + (first_chunk + u) * _C
                hs.append(pltpu.async_copy(dst_hbm.at[pl.ds(b, _C)],
                                           idx_d[u], sem_i))
                hs.append(pltpu.async_copy(src_hbm.at[pl.ds(b, _C)],
                                           idx_s[u], sem_i))
            for h in hs:
                h.wait()
            hs = []
            for u in range(nu):
                hs.append(pltpu.async_copy(q4_hbm.at[idx_d[u]], buf_q[u],
                                           sem_g))
                hs.append(pltpu.async_copy(bk_hbm.at[idx_s[u]], buf_k[u],
                                           sem_g))
                hs.append(pltpu.async_copy(bv_hbm.at[idx_s[u]], buf_v[u],
                                           sem_g))
            for h in hs:
                h.wait()
            hs = []
            for u in range(nu):
                b = wid * per + (first_chunk + u) * _C
                hs.append(pltpu.async_copy(buf_q[u],
                                           gq_hbm.at[pl.ds(b, _C)], sem_w))
                hs.append(pltpu.async_copy(buf_k[u],
                                           gbk_hbm.at[pl.ds(b, _C)], sem_w))
                hs.append(pltpu.async_copy(buf_v[u],
                                           gbv_hbm.at[pl.ds(b, _C)], sem_w))
            for h in hs:
                h.wait()

        def body(i, carry):
            do_group(i * _GU, _GU)
            return carry

        lax.fori_loop(0, nfull, body, 0)
        if ntail:
            do_group(nfull * _GU, ntail)

    @functools.partial(
        pl.kernel,
        mesh=mesh,
        out_type=[
            jax.ShapeDtypeStruct((_NP, 128), jnp.float32),
            jax.ShapeDtypeStruct((_NP, 128), jnp.float32),
        ],
        scratch_types=(
            [pltpu.VMEM((_C,), jnp.int32)] * _SU
            + [pltpu.VMEM((_C, 128), jnp.float32)] * _SU
            + [pltpu.VMEM((_C,), jnp.int32),
               pltpu.VMEM_SHARED((_NP, 128), jnp.float32)]
            + [pltpu.SemaphoreType.DMA] * 2
        ),
    )
    def _scatter2(p0_hbm, p1_hbm, dst_hbm, s0_hbm, s1_hbm, *scr):
        idx_v = scr[:_SU]
        pbuf = scr[_SU:2 * _SU]
        idx_r, acc = scr[2 * _SU], scr[2 * _SU + 1]
        sem_i, sem_s = scr[-2:]
        c = lax.axis_index("c")
        s = lax.axis_index("s")
        row0 = s * _RPT

        def set_idx_r(base):
            for j in range(_C // 16):
                idx_r[pl.ds(j * 16, 16)] = base + j * 16 + lax.iota(
                    jnp.int32, 16)

        def zrow(i, carry):
            for j in range(8):
                pbuf[0][i, pl.ds(j * 16, 16)] = jnp.zeros((16,), jnp.float32)
            return carry

        lax.fori_loop(0, _C, zrow, 0)

        def zacc(i, carry):
            set_idx_r(row0 + i * _C)
            pltpu.sync_copy(pbuf[0], acc.at[idx_r])
            return carry

        lax.fori_loop(0, _RPT // _C, zacc, 0)
        plsc.subcore_barrier()

        per = _E // _NS
        nfull = (per // _C) // _SU
        ntail = (per // _C) % _SU

        def do_group(first_chunk, nu):
            hs = []
            for u in range(nu):
                b = s * per + (first_chunk + u) * _C
                hs.append(pltpu.async_copy(dst_hbm.at[pl.ds(b, _C)],
                                           idx_v[u], sem_i))

            @pl.when(c == 0)
            def _():
                h2 = [pltpu.async_copy(
                    p0_hbm.at[pl.ds(s * per + (first_chunk + u) * _C, _C), :],
                    pbuf[u], sem_s) for u in range(nu)]
                for h in h2:
                    h.wait()

            @pl.when(c == 1)
            def _():
                h2 = [pltpu.async_copy(
                    p1_hbm.at[pl.ds(s * per + (first_chunk + u) * _C, _C), :],
                    pbuf[u], sem_s) for u in range(nu)]
                for h in h2:
                    h.wait()

            for h in hs:
                h.wait()
            hs = []
            for u in range(nu):
                hs.append(pltpu.async_copy(pbuf[u], acc.at[idx_v[u]], sem_s,
                                           add=True))
            for h in hs:
                h.wait()

        def body(i, carry):
            do_group(i * _SU, _SU)
            return carry

        lax.fori_loop(0, nfull, body, 0)
        if ntail:
            do_group(nfull * _SU, ntail)
        plsc.subcore_barrier()

        def wb(i, carry):
            r = row0 + i * _C
            set_idx_r(r)
            pltpu.async_copy(acc.at[idx_r], pbuf[0], sem_i).wait()

            @pl.when(c == 0)
            def _():
                pltpu.sync_copy(pbuf[0], s0_hbm.at[pl.ds(r, _C), :])

            @pl.when(c == 1)
            def _():
                pltpu.sync_copy(pbuf[0], s1_hbm.at[pl.ds(r, _C), :])

            return carry

        lax.fori_loop(0, _RPT // _C, wb, 0)

    mesh1 = plsc.VectorSubcoreMesh(core_axis_name="c", subcore_axis_name="s",
                                   num_cores=1)

    @functools.partial(
        pl.kernel,
        mesh=mesh1,
        out_type=jax.ShapeDtypeStruct((_NP, 128), jnp.float32),
        scratch_types=(
            [pltpu.VMEM((_C,), jnp.int32)] * _SU
            + [pltpu.VMEM((_C, 128), jnp.float32)] * _SU
            + [pltpu.VMEM((_C,), jnp.int32),
               pltpu.VMEM_SHARED((_NP, 128), jnp.float32)]
            + [pltpu.SemaphoreType.DMA] * 2
        ),
    )
    def _scatter_e(eq_hbm, dst_hbm, se_hbm, *scr):
        idx_v = scr[:_SU]
        ebuf = scr[_SU:2 * _SU]
        idx_r, acc = scr[2 * _SU], scr[2 * _SU + 1]
        sem_i, sem_s = scr[-2:]
        s = lax.axis_index("s")
        row0 = s * _RPT

        def set_idx_r(base):
            for j in range(_C // 16):
                idx_r[pl.ds(j * 16, 16)] = base + j * 16 + lax.iota(
                    jnp.int32, 16)

        def zrow(i, carry):
            for j in range(8):
                ebuf[0][i, pl.ds(j * 16, 16)] = jnp.zeros((16,), jnp.float32)
            return carry

        lax.fori_loop(0, _C, zrow, 0)

        def zacc(i, carry):
            set_idx_r(row0 + i * _C)
            pltpu.sync_copy(ebuf[0], acc.at[idx_r])
            return carry

        lax.fori_loop(0, _RPT // _C, zacc, 0)
        plsc.subcore_barrier()

        per = _E // _NS
        nfull = (per // _C) // _SU
        ntail = (per // _C) % _SU

        def do_group(first_chunk, nu):
            hs = []
            for u in range(nu):
                b = s * per + (first_chunk + u) * _C
                hs.append(pltpu.async_copy(dst_hbm.at[pl.ds(b, _C)],
                                           idx_v[u], sem_i))
                hs.append(pltpu.async_copy(eq_hbm.at[pl.ds(b, _C), :],
                                           ebuf[u], sem_s))
            for h in hs:
                h.wait()
            hs = []
            for u in range(nu):
                hs.append(pltpu.async_copy(ebuf[u], acc.at[idx_v[u]], sem_s,
                                           add=True))
            for h in hs:
                h.wait()

        def body(i, carry):
            do_group(i * _SU, _SU)
            return carry

        lax.fori_loop(0, nfull, body, 0)
        if ntail:
            do_group(nfull * _SU, ntail)
        plsc.subcore_barrier()

        def wb(i, carry):
            r = row0 + i * _C
            set_idx_r(r)
            pltpu.async_copy(acc.at[idx_r], ebuf[0], sem_i).wait()
            pltpu.sync_copy(ebuf[0], se_hbm.at[pl.ds(r, _C), :])
            return carry

        lax.fori_loop(0, _RPT // _C, wb, 0)

    return _gather3, _scatter2, _scatter_e


_NP = 10240
_RPT = _NP // _NS


def _block_diag(blk, h):
    return jnp.kron(jnp.eye(h, dtype=jnp.float32), blk.astype(jnp.float32))


def kernel(x_node, x_edge, edge_index, Wq, gq, bq, Wk, gk, bk, Wv, gv, bv,
           Wke, gke, bke, Wve, gve, bve, Wrk, brk, grk_ln, brk_ln,
           Wrv, brv, grv_ln, brv_ln, Wz, bz, gz_ln, bz_ln):
    f32 = jnp.float32
    src = edge_index[0]
    dst = edge_index[1]

    def heads_mat(w):
        return jnp.transpose(w, (1, 0, 2)).reshape(_F, _H * _D).astype(f32)

    M = jnp.asarray(np.kron(np.eye(_H, dtype=np.float32),
                            np.ones((_D, _D), np.float32) / _D))
    Ssum = jnp.asarray(np.kron(np.eye(_H, dtype=np.float32),
                               np.ones((_D, _D), np.float32)))
    sp8 = np.zeros((128, 16), np.float32)
    for h in range(_H):
        sp8[h * _D:(h + 1) * _D, h] = 1.0
    Sp8 = jnp.asarray(sp8)
    rep = np.zeros((16, 128), np.float32)
    for h in range(_H):
        rep[h, h * _D:(h + 1) * _D] = 1.0
    Rep = jnp.asarray(rep)

    nvecs = jnp.stack([gq.reshape(-1), bq.reshape(-1), gk.reshape(-1),
                       bk.reshape(-1), gv.reshape(-1), bv.reshape(-1),
                       jnp.zeros(128, f32), jnp.zeros(128, f32)]).astype(f32)

    evecs = jnp.stack([gke.reshape(-1), bke.reshape(-1),
                       gve.reshape(-1), bve.reshape(-1),
                       jnp.tile(grk_ln, _H), jnp.tile(brk_ln, _H),
                       jnp.tile(grv_ln, _H), jnp.tile(brv_ln, _H),
                       jnp.tile(brk, _H), jnp.tile(brv, _H)]
                      + [jnp.zeros(128, f32)] * 6).astype(f32)

    fvecs = jnp.stack([jnp.tile(gz_ln, _H), jnp.tile(bz_ln, _H),
                       jnp.tile(bz, _H)]
                      + [jnp.zeros(128, f32)] * 5).astype(f32)

    wmat = pl.BlockSpec((128, 128), lambda i: (0, 0))
    vspec8 = pl.BlockSpec((8, 128), lambda i: (0, 0))
    vspec16 = pl.BlockSpec((16, 128), lambda i: (0, 0))

    nb = _N // _BN
    nblk = pl.BlockSpec((_BN, 128), lambda i: (i, 0))
    q4, bkn, bvn = pl.pallas_call(
        _node_body,
        grid=(nb,),
        in_specs=[nblk, wmat, wmat, wmat, wmat, wmat, wmat, vspec8],
        out_specs=[nblk, nblk, nblk],
        out_shape=[jax.ShapeDtypeStruct((_N, 128), f32)] * 3,
    )(x_node.astype(f32), heads_mat(Wq), heads_mat(Wk), heads_mat(Wv),
      _block_diag(Wrk[_D:], _H), _block_diag(Wrv[_D:], _H), M, nvecs)

    gather3, scatter2, scatter_e = _sc_kernels()
    gqv, gbk, gbv = gather3(q4, bkn, bvn, src, dst)

    eb = _E // _BE
    eblk = pl.BlockSpec((_BE, 128), lambda i: (i, 0))
    p0, p1, eq = pl.pallas_call(
        _edge_body,
        grid=(eb,),
        in_specs=[eblk, eblk, eblk, eblk, wmat, wmat, wmat, wmat, wmat,
                  wmat, vspec16],
        out_specs=[eblk, eblk, eblk],
        out_shape=[jax.ShapeDtypeStruct((_E, 128), f32)] * 3,
    )(x_edge.astype(f32), gqv, gbk, gbv, heads_mat(Wke), heads_mat(Wve),
      _block_diag(Wrk[:_D], _H), _block_diag(Wrv[:_D], _H), M, Ssum,
      evecs)

    s0p, s1p = scatter2(p0, p1, dst)
    sep = scatter_e(eq, dst)
    s0, s1, sexp = s0p[:_N], s1p[:_N], sep[:_N]

    out = pl.pallas_call(
        _final_body,
        grid=(nb,),
        in_specs=[nblk, nblk, nblk, wmat, wmat, vspec8],
        out_specs=nblk,
        out_shape=jax.ShapeDtypeStruct((_N, 128), f32),
    )(s0, s1, sexp, _block_diag(Wz, _H), M, fvecs)

    return out
